# Initial kernel scaffold; baseline (speedup 1.0000x reference)
import jax
import jax.numpy as jnp
from jax.experimental import pallas as pl


def kernel(src_na, dst_na, ea, edge_idx, Wv1, bv1, Wv2, bv2, Wv3, bv3, Wb1, bb1, Wb2, bb2, Wb3, bb3, Wo):
    raise NotImplementedError("write your pallas kernel here")


# speedup vs baseline: 15.4356x; 15.4356x over previous
import functools
import math

import jax
import jax.numpy as jnp
from jax import lax
from jax.experimental import pallas as pl
from jax.experimental.pallas import tpu as pltpu
from jax.experimental.pallas import tpu_sc as plsc

_NC = 2
_NS = 16
_C = 80


def _sc_gather(src_na, dst_na, sidx, didx):
    n, h = src_na.shape
    e = sidx.shape[0]
    nw = _NC * _NS
    ew = e // nw
    iters = ew // _C
    mesh = plsc.VectorSubcoreMesh(core_axis_name="c", subcore_axis_name="s")

    @functools.partial(
        pl.kernel,
        out_type=(jax.ShapeDtypeStruct((e, h), jnp.float32),
                  jax.ShapeDtypeStruct((e, h), jnp.float32)),
        mesh=mesh,
        scratch_types=[
            pltpu.VMEM((_C,), jnp.int32),
            pltpu.VMEM((_C,), jnp.int32),
            pltpu.VMEM((_C, h), jnp.float32),
            pltpu.VMEM((_C, h), jnp.float32),
            pltpu.SemaphoreType.DMA,
            pltpu.SemaphoreType.DMA,
        ],
    )
    def k(src_hbm, dst_hbm, sidx_hbm, didx_hbm, osrc_hbm, odst_hbm,
          siv, div, srows, drows, sem1, sem2):
        wid = lax.axis_index("s") * _NC + lax.axis_index("c")
        wbase = wid * ew

        def body(i, carry):
            base = wbase + i * _C
            pltpu.sync_copy(sidx_hbm.at[pl.ds(base, _C)], siv)
            pltpu.sync_copy(didx_hbm.at[pl.ds(base, _C)], div)
            cp1 = pltpu.async_copy(src_hbm.at[siv], srows, sem1)
            cp2 = pltpu.async_copy(dst_hbm.at[div], drows, sem2)
            cp1.wait()
            cp2.wait()
            pltpu.sync_copy(srows, osrc_hbm.at[pl.ds(base, _C)])
            pltpu.sync_copy(drows, odst_hbm.at[pl.ds(base, _C)])
            return carry

        lax.fori_loop(0, iters, body, 0)

    return k(src_na, dst_na, sidx, didx)


def _sc_scatter(didx, evx, n):
    e = evx.shape[1]
    ng = evx.shape[0] // 8
    cc = 1280
    iters = e // cc
    groups = cc // 16
    zc = 16000
    zi = (n * 8) // zc
    mesh = plsc.VectorSubcoreMesh(core_axis_name="c", subcore_axis_name="s")

    @functools.partial(
        pl.kernel,
        out_type=jax.ShapeDtypeStruct((ng, n * 8), jnp.float32),
        mesh=mesh,
        compiler_params=pltpu.CompilerParams(needs_layout_passes=False),
        scratch_types=[
            pltpu.VMEM((cc,), jnp.int32),
            pltpu.VMEM((8, cc), jnp.float32),
            pltpu.VMEM((n * 8,), jnp.float32),
        ],
    )
    def k(didx_hbm, evx_hbm, out_hbm, idxv, vbuf, acc):
        c = lax.axis_index("c")
        s = lax.axis_index("s")
        w = s * _NC + c

        @pl.when(w < ng)
        def _():
            zeros16 = jnp.zeros((16,), jnp.float32)

            def initb(j, carry):
                acc[pl.ds(j * 16, 16)] = zeros16
                return carry

            lax.fori_loop(0, (n * 8) // 16, initb, 0)

            def body(i, carry):
                base = i * cc
                pltpu.sync_copy(didx_hbm.at[pl.ds(base, cc)], idxv)
                pltpu.sync_copy(evx_hbm.at[pl.ds(w * 8, 8), pl.ds(base, cc)],
                                vbuf)

                def grp(g, carry2):
                    nidx8 = idxv[pl.ds(g * 16, 16)] * 8
                    for kk in range(8):
                        vals = vbuf[kk, pl.ds(g * 16, 16)]
                        plsc.addupdate_scatter(acc, [nidx8 + kk], vals)
                    return carry2

                lax.fori_loop(0, groups, grp, 0)
                return carry

            lax.fori_loop(0, iters, body, 0)

            def drainb(j, carry):
                pltpu.sync_copy(acc.at[pl.ds(j * zc, zc)],
                                out_hbm.at[w, pl.ds(j * zc, zc)])
                return carry

            lax.fori_loop(0, zi, drainb, 0)

    return k(didx, evx)


def _gelu(x):
    return 0.5 * x * (1.0 + lax.erf(x * (1.0 / math.sqrt(2.0))))


def _full_spec(shape):
    return pl.BlockSpec(shape, lambda i: tuple(0 for _ in shape))


def _tc_fused(srows, drows, ea, wb1s, wb1e, wb1d, b1, wb2, b2, wb3, b3,
              wv1s, wv1e, c1, wv2, c2, wv3, c3, heads):
    e, h = ea.shape
    eb = 1280
    grid = e // eb
    d = h // heads
    inv = 1.0 / math.sqrt(d)

    def body(xs_ref, xd_ref, xe_ref, wb1s_r, wb1e_r, wb1d_r, b1_r, wb2_r,
             b2_r, wb3_r, b3_r, wv1s_r, wv1e_r, c1_r, wv2_r, c2_r, wv3_r,
             c3_r, evx_ref):
        xs = xs_ref[...]
        xd = xd_ref[...]
        xe = xe_ref[...]
        hb = xs @ wb1s_r[...] + xe @ wb1e_r[...] + xd @ wb1d_r[...] + b1_r[...]
        hb = jnp.maximum(hb, 0.0)
        hb = jnp.maximum(hb @ wb2_r[...] + b2_r[...], 0.0)
        w = (hb @ wb3_r[...] + b3_r[...]) * inv
        ex = jnp.exp(jnp.clip(w, -60.0, 60.0))
        hv = _gelu(xs @ wv1s_r[...] + xe @ wv1e_r[...] + c1_r[...])
        hv = _gelu(hv @ wv2_r[...] + c2_r[...])
        v = hv @ wv3_r[...] + c3_r[...]
        hr = lax.broadcasted_iota(jnp.int32, (heads, h), 0)
        lh = lax.broadcasted_iota(jnp.int32, (heads, h), 1) // d
        ev = v * (ex @ jnp.where(hr == lh, 1.0, 0.0))
        i0 = lax.broadcasted_iota(jnp.int32, (heads, 8), 0)
        i1 = lax.broadcasted_iota(jnp.int32, (heads, 8), 1)
        ep8 = ex @ jnp.where(i0 == i1, 1.0, 0.0)
        evx_ref[...] = jnp.concatenate([ev.T, ep8.T], axis=0)

    weights = [wb1s, wb1e, wb1d, b1, wb2, b2, wb3, b3,
               wv1s, wv1e, c1, wv2, c2, wv3, c3]
    return pl.pallas_call(
        body,
        grid=(grid,),
        in_specs=[pl.BlockSpec((eb, h), lambda i: (i, 0))] * 3
                 + [_full_spec(x.shape) for x in weights],
        out_specs=pl.BlockSpec((h + 8, eb), lambda i: (0, i)),
        out_shape=jax.ShapeDtypeStruct((h + 8, e), jnp.float32),
    )(srows, drows, ea, *weights)


def _tc_final(slabs, wo_t, heads):
    ng, n, _ = slabs.shape
    h = wo_t.shape[0]
    d = h // heads
    nb = 2000
    grid = n // nb

    def body(n_ref, wo_ref, o_ref):
        num = jnp.zeros((nb, h), jnp.float32)
        for t in range(h // 8):
            jr = lax.broadcasted_iota(jnp.int32, (8, h), 0)
            lr = lax.broadcasted_iota(jnp.int32, (8, h), 1)
            sel = jnp.where(lr == 8 * t + jr, 1.0, 0.0)
            num = num + n_ref[t] @ sel
        s8 = n_ref[h // 8]
        j8 = lax.broadcasted_iota(jnp.int32, (8, h), 0)
        l8 = lax.broadcasted_iota(jnp.int32, (8, h), 1) // d
        srep = s8 @ jnp.where(j8 == l8, 1.0, 0.0) + 1e-16
        o_ref[...] = (num / srep) @ wo_ref[...]

    return pl.pallas_call(
        body,
        grid=(grid,),
        in_specs=[pl.BlockSpec((ng, nb, 8), lambda i: (0, i, 0)),
                  _full_spec((h, h))],
        out_specs=pl.BlockSpec((nb, h), lambda i: (i, 0)),
        out_shape=jax.ShapeDtypeStruct((n, h), jnp.float32),
    )(slabs, wo_t)


def kernel(src_na, dst_na, ea, edge_idx, Wv1, bv1, Wv2, bv2, Wv3, bv3,
           Wb1, bb1, Wb2, bb2, Wb3, bb3, Wo):
    n, h = src_na.shape
    heads = Wb3.shape[0]
    dst_idx = edge_idx[0]
    src_idx = edge_idx[1]

    srows, drows = _sc_gather(src_na, dst_na, src_idx, dst_idx)

    evx = _tc_fused(
        srows, drows, ea,
        Wb1[:, :h].T, Wb1[:, h:2 * h].T, Wb1[:, 2 * h:].T, bb1.reshape(1, -1),
        Wb2.T, bb2.reshape(1, -1), Wb3.T, bb3.reshape(1, -1),
        Wv1[:, :h].T, Wv1[:, h:].T, bv1.reshape(1, -1),
        Wv2.T, bv2.reshape(1, -1), Wv3.T, bv3.reshape(1, -1), heads)

    slabs = _sc_scatter(dst_idx, evx, n).reshape(-1, n, 8)
    return _tc_final(slabs, Wo.T, heads)


# speedup vs baseline: 18.3209x; 1.1869x over previous
import functools
import math

import jax
import jax.numpy as jnp
from jax import lax
from jax.experimental import pallas as pl
from jax.experimental.pallas import tpu as pltpu
from jax.experimental.pallas import tpu_sc as plsc

_NC = 2
_NS = 16
_C = 80


def _sc_gather(src_na, dst_na, sidx, didx):
    n, h = src_na.shape
    e = sidx.shape[0]
    nw = _NC * _NS
    ew = e // nw
    iters = ew // _C
    mesh = plsc.VectorSubcoreMesh(core_axis_name="c", subcore_axis_name="s")

    @functools.partial(
        pl.kernel,
        out_type=(jax.ShapeDtypeStruct((e, h), jnp.float32),
                  jax.ShapeDtypeStruct((e, h), jnp.float32)),
        mesh=mesh,
        scratch_types=[
            pltpu.VMEM((_C,), jnp.int32),
            pltpu.VMEM((_C,), jnp.int32),
            pltpu.VMEM((_C, h), jnp.float32),
            pltpu.VMEM((_C, h), jnp.float32),
            pltpu.SemaphoreType.DMA,
            pltpu.SemaphoreType.DMA,
        ],
    )
    def k(src_hbm, dst_hbm, sidx_hbm, didx_hbm, osrc_hbm, odst_hbm,
          siv, div, srows, drows, sem1, sem2):
        wid = lax.axis_index("s") * _NC + lax.axis_index("c")
        wbase = wid * ew

        def body(i, carry):
            base = wbase + i * _C
            pltpu.sync_copy(sidx_hbm.at[pl.ds(base, _C)], siv)
            pltpu.sync_copy(didx_hbm.at[pl.ds(base, _C)], div)
            cp1 = pltpu.async_copy(src_hbm.at[siv], srows, sem1)
            cp2 = pltpu.async_copy(dst_hbm.at[div], drows, sem2)
            cp1.wait()
            cp2.wait()
            pltpu.sync_copy(srows, osrc_hbm.at[pl.ds(base, _C)])
            pltpu.sync_copy(drows, odst_hbm.at[pl.ds(base, _C)])
            return carry

        lax.fori_loop(0, iters, body, 0)

    return k(src_na, dst_na, sidx, didx)


def _sc_scatter(didx, evx, n):
    e = evx.shape[1]
    ng = evx.shape[0] // 8
    cc = 1280
    iters = e // cc
    groups = cc // 16
    zc = 16000
    zi = (n * 8) // zc
    mesh = plsc.VectorSubcoreMesh(core_axis_name="c", subcore_axis_name="s")

    assert iters % 2 == 0

    @functools.partial(
        pl.kernel,
        out_type=jax.ShapeDtypeStruct((ng, n * 8), jnp.float32),
        mesh=mesh,
        compiler_params=pltpu.CompilerParams(needs_layout_passes=False),
        scratch_types=[
            pltpu.VMEM((cc,), jnp.int32),
            pltpu.VMEM((cc,), jnp.int32),
            pltpu.VMEM((8, cc), jnp.float32),
            pltpu.VMEM((8, cc), jnp.float32),
            pltpu.VMEM((n * 8,), jnp.float32),
            pltpu.SemaphoreType.DMA,
            pltpu.SemaphoreType.DMA,
        ],
    )
    def k(didx_hbm, evx_hbm, out_hbm, idxv0, idxv1, vbuf0, vbuf1, acc,
          sem0, sem1):
        c = lax.axis_index("c")
        s = lax.axis_index("s")
        w = s * _NC + c

        @pl.when(w < ng)
        def _():
            zeros16 = jnp.zeros((16,), jnp.float32)

            def initb(j, carry):
                acc[pl.ds(j * 16, 16)] = zeros16
                return carry

            lax.fori_loop(0, (n * 8) // 16, initb, 0)

            def start(i, idxv, vbuf, sem):
                base = i * cc
                pltpu.async_copy(didx_hbm.at[pl.ds(base, cc)], idxv, sem)
                pltpu.async_copy(evx_hbm.at[pl.ds(w * 8, 8),
                                            pl.ds(base, cc)], vbuf, sem)

            def wait(idxv, vbuf, sem):
                pltpu.make_async_copy(didx_hbm.at[pl.ds(0, cc)], idxv,
                                      sem).wait()
                pltpu.make_async_copy(evx_hbm.at[pl.ds(0, 8), pl.ds(0, cc)],
                                      vbuf, sem).wait()

            def process(idxv, vbuf):
                def grp(g, carry2):
                    nidx8 = idxv[pl.ds(g * 16, 16)] * 8
                    for kk in range(8):
                        plsc.addupdate_scatter(
                            acc, [nidx8 + kk], vbuf[kk, pl.ds(g * 16, 16)])
                    return carry2

                lax.fori_loop(0, groups, grp, 0)

            start(0, idxv0, vbuf0, sem0)

            def body(i2, carry):
                i = i2 * 2
                wait(idxv0, vbuf0, sem0)

                @pl.when(i + 1 < iters)
                def _():
                    start(i + 1, idxv1, vbuf1, sem1)

                process(idxv0, vbuf0)

                @pl.when(i + 1 < iters)
                def _():
                    wait(idxv1, vbuf1, sem1)

                    @pl.when(i + 2 < iters)
                    def _():
                        start(i + 2, idxv0, vbuf0, sem0)

                    process(idxv1, vbuf1)

                return carry

            lax.fori_loop(0, iters // 2, body, 0)

            def drainb(j, carry):
                pltpu.sync_copy(acc.at[pl.ds(j * zc, zc)],
                                out_hbm.at[w, pl.ds(j * zc, zc)])
                return carry

            lax.fori_loop(0, zi, drainb, 0)

    return k(didx, evx)


def _gelu(x):
    return 0.5 * x * (1.0 + lax.erf(x * (1.0 / math.sqrt(2.0))))


def _full_spec(shape):
    return pl.BlockSpec(shape, lambda i: tuple(0 for _ in shape))


def _tc_fused(srows, drows, ea, wb1s, wb1e, wb1d, b1, wb2, b2, wb3, b3,
              wv1s, wv1e, c1, wv2, c2, wv3, c3, heads):
    e, h = ea.shape
    eb = 1280
    grid = e // eb
    d = h // heads
    inv = 1.0 / math.sqrt(d)

    def body(xs_ref, xd_ref, xe_ref, wb1s_r, wb1e_r, wb1d_r, b1_r, wb2_r,
             b2_r, wb3_r, b3_r, wv1s_r, wv1e_r, c1_r, wv2_r, c2_r, wv3_r,
             c3_r, evx_ref):
        xs = xs_ref[...]
        xd = xd_ref[...]
        xe = xe_ref[...]
        hb = xs @ wb1s_r[...] + xe @ wb1e_r[...] + xd @ wb1d_r[...] + b1_r[...]
        hb = jnp.maximum(hb, 0.0)
        hb = jnp.maximum(hb @ wb2_r[...] + b2_r[...], 0.0)
        w = (hb @ wb3_r[...] + b3_r[...]) * inv
        ex = jnp.exp(jnp.clip(w, -60.0, 60.0))
        hv = _gelu(xs @ wv1s_r[...] + xe @ wv1e_r[...] + c1_r[...])
        hv = _gelu(hv @ wv2_r[...] + c2_r[...])
        v = hv @ wv3_r[...] + c3_r[...]
        hr = lax.broadcasted_iota(jnp.int32, (heads, h), 0)
        lh = lax.broadcasted_iota(jnp.int32, (heads, h), 1) // d
        ev = v * (ex @ jnp.where(hr == lh, 1.0, 0.0))
        i0 = lax.broadcasted_iota(jnp.int32, (heads, 8), 0)
        i1 = lax.broadcasted_iota(jnp.int32, (heads, 8), 1)
        ep8 = ex @ jnp.where(i0 == i1, 1.0, 0.0)
        evx_ref[...] = jnp.concatenate([ev.T, ep8.T], axis=0)

    weights = [wb1s, wb1e, wb1d, b1, wb2, b2, wb3, b3,
               wv1s, wv1e, c1, wv2, c2, wv3, c3]
    return pl.pallas_call(
        body,
        grid=(grid,),
        in_specs=[pl.BlockSpec((eb, h), lambda i: (i, 0))] * 3
                 + [_full_spec(x.shape) for x in weights],
        out_specs=pl.BlockSpec((h + 8, eb), lambda i: (0, i)),
        out_shape=jax.ShapeDtypeStruct((h + 8, e), jnp.float32),
    )(srows, drows, ea, *weights)


def _tc_final(slabs, wo_t, heads):
    ng, n, _ = slabs.shape
    h = wo_t.shape[0]
    d = h // heads
    nb = 2000
    grid = n // nb

    def body(n_ref, wo_ref, o_ref):
        num = jnp.zeros((nb, h), jnp.float32)
        for t in range(h // 8):
            jr = lax.broadcasted_iota(jnp.int32, (8, h), 0)
            lr = lax.broadcasted_iota(jnp.int32, (8, h), 1)
            sel = jnp.where(lr == 8 * t + jr, 1.0, 0.0)
            num = num + n_ref[t] @ sel
        s8 = n_ref[h // 8]
        j8 = lax.broadcasted_iota(jnp.int32, (8, h), 0)
        l8 = lax.broadcasted_iota(jnp.int32, (8, h), 1) // d
        srep = s8 @ jnp.where(j8 == l8, 1.0, 0.0) + 1e-16
        o_ref[...] = (num / srep) @ wo_ref[...]

    return pl.pallas_call(
        body,
        grid=(grid,),
        in_specs=[pl.BlockSpec((ng, nb, 8), lambda i: (0, i, 0)),
                  _full_spec((h, h))],
        out_specs=pl.BlockSpec((nb, h), lambda i: (i, 0)),
        out_shape=jax.ShapeDtypeStruct((n, h), jnp.float32),
    )(slabs, wo_t)


def kernel(src_na, dst_na, ea, edge_idx, Wv1, bv1, Wv2, bv2, Wv3, bv3,
           Wb1, bb1, Wb2, bb2, Wb3, bb3, Wo):
    n, h = src_na.shape
    heads = Wb3.shape[0]
    dst_idx = edge_idx[0]
    src_idx = edge_idx[1]

    srows, drows = _sc_gather(src_na, dst_na, src_idx, dst_idx)

    evx = _tc_fused(
        srows, drows, ea,
        Wb1[:, :h].T, Wb1[:, h:2 * h].T, Wb1[:, 2 * h:].T, bb1.reshape(1, -1),
        Wb2.T, bb2.reshape(1, -1), Wb3.T, bb3.reshape(1, -1),
        Wv1[:, :h].T, Wv1[:, h:].T, bv1.reshape(1, -1),
        Wv2.T, bv2.reshape(1, -1), Wv3.T, bv3.reshape(1, -1), heads)

    slabs = _sc_scatter(dst_idx, evx, n).reshape(-1, n, 8)
    return _tc_final(slabs, Wo.T, heads)


# speedup vs baseline: 23.9383x; 1.3066x over previous
import functools
import math

import jax
import jax.numpy as jnp
from jax import lax
from jax.experimental import pallas as pl
from jax.experimental.pallas import tpu as pltpu
from jax.experimental.pallas import tpu_sc as plsc

_NC = 2
_NS = 16
_C = 80


def _sc_gather(src_na, dst_na, sidx, didx):
    n, h = src_na.shape
    e = sidx.shape[0]
    nw = _NC * _NS
    ew = e // nw
    iters = ew // _C
    mesh = plsc.VectorSubcoreMesh(core_axis_name="c", subcore_axis_name="s")

    @functools.partial(
        pl.kernel,
        out_type=(jax.ShapeDtypeStruct((e, h), jnp.float32),
                  jax.ShapeDtypeStruct((e, h), jnp.float32)),
        mesh=mesh,
        scratch_types=[
            pltpu.VMEM((_C,), jnp.int32),
            pltpu.VMEM((_C,), jnp.int32),
            pltpu.VMEM((_C, h), jnp.float32),
            pltpu.VMEM((_C, h), jnp.float32),
            pltpu.SemaphoreType.DMA,
            pltpu.SemaphoreType.DMA,
        ],
    )
    def k(src_hbm, dst_hbm, sidx_hbm, didx_hbm, osrc_hbm, odst_hbm,
          siv, div, srows, drows, sem1, sem2):
        wid = lax.axis_index("s") * _NC + lax.axis_index("c")
        wbase = wid * ew

        def body(i, carry):
            base = wbase + i * _C
            pltpu.sync_copy(sidx_hbm.at[pl.ds(base, _C)], siv)
            pltpu.sync_copy(didx_hbm.at[pl.ds(base, _C)], div)
            cp1 = pltpu.async_copy(src_hbm.at[siv], srows, sem1)
            cp2 = pltpu.async_copy(dst_hbm.at[div], drows, sem2)
            cp1.wait()
            cp2.wait()
            pltpu.sync_copy(srows, osrc_hbm.at[pl.ds(base, _C)])
            pltpu.sync_copy(drows, odst_hbm.at[pl.ds(base, _C)])
            return carry

        lax.fori_loop(0, iters, body, 0)

    return k(src_na, dst_na, sidx, didx)


def _sc_scatter(didx, evx, n):
    e = evx.shape[1]
    ng = evx.shape[0] // 8
    cc = 1280
    iters = e // cc
    groups = cc // 16
    zc = 16000
    zi = (n * 8) // zc
    mesh = plsc.VectorSubcoreMesh(core_axis_name="c", subcore_axis_name="s")

    assert iters % 2 == 0 and groups % 2 == 0

    @functools.partial(
        pl.kernel,
        out_type=jax.ShapeDtypeStruct((ng, 8, n), jnp.float32),
        mesh=mesh,
        compiler_params=pltpu.CompilerParams(needs_layout_passes=False),
        scratch_types=[
            pltpu.VMEM((cc,), jnp.int32),
            pltpu.VMEM((cc,), jnp.int32),
            pltpu.VMEM((8, cc), jnp.float32),
            pltpu.VMEM((8, cc), jnp.float32),
        ] + [pltpu.VMEM((n,), jnp.float32) for _ in range(8)] + [
            pltpu.SemaphoreType.DMA,
            pltpu.SemaphoreType.DMA,
        ],
    )
    def k(didx_hbm, evx_hbm, out_hbm, idxv0, idxv1, vbuf0, vbuf1,
          a0, a1, a2, a3, a4, a5, a6, a7, sem0, sem1):
        accs = (a0, a1, a2, a3, a4, a5, a6, a7)
        c = lax.axis_index("c")
        s = lax.axis_index("s")
        w = s * _NC + c

        @pl.when(w < ng)
        def _():
            zeros16 = jnp.zeros((16,), jnp.float32)

            def initb(j, carry):
                for kk in range(8):
                    accs[kk][pl.ds(j * 16, 16)] = zeros16
                return carry

            lax.fori_loop(0, n // 16, initb, 0)

            def start(i, idxv, vbuf, sem):
                base = i * cc
                pltpu.async_copy(didx_hbm.at[pl.ds(base, cc)], idxv, sem)
                pltpu.async_copy(evx_hbm.at[pl.ds(w * 8, 8),
                                            pl.ds(base, cc)], vbuf, sem)

            def wait(idxv, vbuf, sem):
                pltpu.make_async_copy(didx_hbm.at[pl.ds(0, cc)], idxv,
                                      sem).wait()
                pltpu.make_async_copy(evx_hbm.at[pl.ds(0, 8), pl.ds(0, cc)],
                                      vbuf, sem).wait()

            def process(idxv, vbuf):
                def grp(g2, carry2):
                    for u in range(2):
                        g16 = (g2 * 2 + u) * 16
                        nidx = idxv[pl.ds(g16, 16)]
                        for kk in range(8):
                            plsc.addupdate_scatter(
                                accs[kk], [nidx], vbuf[kk, pl.ds(g16, 16)])
                    return carry2

                lax.fori_loop(0, groups // 2, grp, 0)

            start(0, idxv0, vbuf0, sem0)

            def body(i2, carry):
                i = i2 * 2
                wait(idxv0, vbuf0, sem0)

                @pl.when(i + 1 < iters)
                def _():
                    start(i + 1, idxv1, vbuf1, sem1)

                process(idxv0, vbuf0)

                @pl.when(i + 1 < iters)
                def _():
                    wait(idxv1, vbuf1, sem1)

                    @pl.when(i + 2 < iters)
                    def _():
                        start(i + 2, idxv0, vbuf0, sem0)

                    process(idxv1, vbuf1)

                return carry

            lax.fori_loop(0, iters // 2, body, 0)

            for kk in range(8):
                pltpu.sync_copy(accs[kk], out_hbm.at[w, kk])

    return k(didx, evx)


def _gelu(x):
    return 0.5 * x * (1.0 + lax.erf(x * (1.0 / math.sqrt(2.0))))


def _full_spec(shape):
    return pl.BlockSpec(shape, lambda i: tuple(0 for _ in shape))


def _tc_fused(srows, drows, ea, wb1s, wb1e, wb1d, b1, wb2, b2, wb3, b3,
              wv1s, wv1e, c1, wv2, c2, wv3, c3, heads):
    e, h = ea.shape
    eb = 1280
    grid = e // eb
    d = h // heads
    inv = 1.0 / math.sqrt(d)

    def body(xs_ref, xd_ref, xe_ref, wb1s_r, wb1e_r, wb1d_r, b1_r, wb2_r,
             b2_r, wb3_r, b3_r, wv1s_r, wv1e_r, c1_r, wv2_r, c2_r, wv3_r,
             c3_r, evx_ref):
        xs = xs_ref[...]
        xd = xd_ref[...]
        xe = xe_ref[...]
        hb = xs @ wb1s_r[...] + xe @ wb1e_r[...] + xd @ wb1d_r[...] + b1_r[...]
        hb = jnp.maximum(hb, 0.0)
        hb = jnp.maximum(hb @ wb2_r[...] + b2_r[...], 0.0)
        w = (hb @ wb3_r[...] + b3_r[...]) * inv
        ex = jnp.exp(jnp.clip(w, -60.0, 60.0))
        hv = _gelu(xs @ wv1s_r[...] + xe @ wv1e_r[...] + c1_r[...])
        hv = _gelu(hv @ wv2_r[...] + c2_r[...])
        v = hv @ wv3_r[...] + c3_r[...]
        hr = lax.broadcasted_iota(jnp.int32, (heads, h), 0)
        lh = lax.broadcasted_iota(jnp.int32, (heads, h), 1) // d
        ev = v * (ex @ jnp.where(hr == lh, 1.0, 0.0))
        i0 = lax.broadcasted_iota(jnp.int32, (heads, 8), 0)
        i1 = lax.broadcasted_iota(jnp.int32, (heads, 8), 1)
        ep8 = ex @ jnp.where(i0 == i1, 1.0, 0.0)
        evx_ref[...] = jnp.concatenate([ev.T, ep8.T], axis=0)

    weights = [wb1s, wb1e, wb1d, b1, wb2, b2, wb3, b3,
               wv1s, wv1e, c1, wv2, c2, wv3, c3]
    return pl.pallas_call(
        body,
        grid=(grid,),
        in_specs=[pl.BlockSpec((eb, h), lambda i: (i, 0))] * 3
                 + [_full_spec(x.shape) for x in weights],
        out_specs=pl.BlockSpec((h + 8, eb), lambda i: (0, i)),
        out_shape=jax.ShapeDtypeStruct((h + 8, e), jnp.float32),
    )(srows, drows, ea, *weights)


def _tc_final(slabs, wo, heads):
    n = slabs.shape[1]
    h = wo.shape[0]
    d = h // heads

    def body(sl_ref, wo_ref, o_ref):
        allr = sl_ref[...]
        numt = allr[:h, :]
        s8t = allr[h:h + 8, :]
        lr = lax.broadcasted_iota(jnp.int32, (h, 8), 0) // d
        jr = lax.broadcasted_iota(jnp.int32, (h, 8), 1)
        srept = jnp.where(jr == lr, 1.0, 0.0) @ s8t + 1e-16
        o_ref[...] = (wo_ref[...] @ (numt / srept)).T

    return pl.pallas_call(
        body,
        grid=(1,),
        in_specs=[_full_spec((h + 8, n)), _full_spec((h, h))],
        out_specs=pl.BlockSpec((n, h), lambda i: (0, 0)),
        out_shape=jax.ShapeDtypeStruct((n, h), jnp.float32),
    )(slabs, wo)


def kernel(src_na, dst_na, ea, edge_idx, Wv1, bv1, Wv2, bv2, Wv3, bv3,
           Wb1, bb1, Wb2, bb2, Wb3, bb3, Wo):
    n, h = src_na.shape
    heads = Wb3.shape[0]
    dst_idx = edge_idx[0]
    src_idx = edge_idx[1]

    srows, drows = _sc_gather(src_na, dst_na, src_idx, dst_idx)

    evx = _tc_fused(
        srows, drows, ea,
        Wb1[:, :h].T, Wb1[:, h:2 * h].T, Wb1[:, 2 * h:].T, bb1.reshape(1, -1),
        Wb2.T, bb2.reshape(1, -1), Wb3.T, bb3.reshape(1, -1),
        Wv1[:, :h].T, Wv1[:, h:].T, bv1.reshape(1, -1),
        Wv2.T, bv2.reshape(1, -1), Wv3.T, bv3.reshape(1, -1), heads)

    slabs = _sc_scatter(dst_idx, evx, n).reshape(-1, n)
    return _tc_final(slabs, Wo, heads)


# speedup vs baseline: 26.4261x; 1.1039x over previous
import functools
import math

import jax
import jax.numpy as jnp
from jax import lax
from jax.experimental import pallas as pl
from jax.experimental.pallas import tpu as pltpu
from jax.experimental.pallas import tpu_sc as plsc

_NC = 2
_NS = 16
_C = 80


def _sc_gather(src_na, dst_na, sidx, didx):
    n, h = src_na.shape
    e = sidx.shape[0]
    nw = _NC * _NS
    ew = e // nw
    iters = ew // _C
    mesh = plsc.VectorSubcoreMesh(core_axis_name="c", subcore_axis_name="s")

    @functools.partial(
        pl.kernel,
        out_type=(jax.ShapeDtypeStruct((e, h), jnp.float32),
                  jax.ShapeDtypeStruct((e, h), jnp.float32)),
        mesh=mesh,
        scratch_types=[
            pltpu.VMEM((_C,), jnp.int32),
            pltpu.VMEM((_C,), jnp.int32),
            pltpu.VMEM((_C, h), jnp.float32),
            pltpu.VMEM((_C, h), jnp.float32),
            pltpu.VMEM((_C,), jnp.int32),
            pltpu.VMEM((_C,), jnp.int32),
            pltpu.VMEM((_C, h), jnp.float32),
            pltpu.VMEM((_C, h), jnp.float32),
            pltpu.SemaphoreType.DMA,
            pltpu.SemaphoreType.DMA,
            pltpu.SemaphoreType.DMA,
            pltpu.SemaphoreType.DMA,
            pltpu.SemaphoreType.DMA,
            pltpu.SemaphoreType.DMA,
        ],
    )
    def k(src_hbm, dst_hbm, sidx_hbm, didx_hbm, osrc_hbm, odst_hbm,
          siv0, div0, srows0, drows0, siv1, div1, srows1, drows1,
          isem0, gsem0, osem0, isem1, gsem1, osem1):
        wid = lax.axis_index("s") * _NC + lax.axis_index("c")
        wbase = wid * ew
        bufs = ((siv0, div0, srows0, drows0, isem0, gsem0, osem0),
                (siv1, div1, srows1, drows1, isem1, gsem1, osem1))

        def start_idx(i, b):
            siv, div, _, _, isem, _, _ = bufs[b]
            base = wbase + i * _C
            pltpu.async_copy(sidx_hbm.at[pl.ds(base, _C)], siv, isem)
            pltpu.async_copy(didx_hbm.at[pl.ds(base, _C)], div, isem)

        def phase(i, b, first, last):
            siv, div, srows, drows, isem, gsem, osem = bufs[b]
            base = wbase + i * _C
            if not first:
                pltpu.make_async_copy(srows, osrc_hbm.at[pl.ds(0, _C)],
                                      osem).wait()
                pltpu.make_async_copy(drows, odst_hbm.at[pl.ds(0, _C)],
                                      osem).wait()
            pltpu.make_async_copy(sidx_hbm.at[pl.ds(0, _C)], siv, isem).wait()
            pltpu.make_async_copy(didx_hbm.at[pl.ds(0, _C)], div, isem).wait()
            cp1 = pltpu.async_copy(src_hbm.at[siv], srows, gsem)
            cp2 = pltpu.async_copy(dst_hbm.at[div], drows, gsem)
            cp1.wait()
            cp2.wait()
            if not last:
                start_idx(i + 2, b)
            pltpu.async_copy(srows, osrc_hbm.at[pl.ds(base, _C)], osem)
            pltpu.async_copy(drows, odst_hbm.at[pl.ds(base, _C)], osem)

        assert iters % 2 == 1 and iters >= 5
        start_idx(0, 0)
        start_idx(1, 1)
        phase(0, 0, True, False)
        phase(1, 1, True, False)

        def body(i2, carry):
            i = 2 + i2 * 2
            phase(i, 0, False, False)
            phase(i + 1, 1, False, False)
            return carry

        lax.fori_loop(0, (iters - 5) // 2, body, 0)
        phase(iters - 3, 0, False, False)
        phase(iters - 2, 1, False, True)
        phase(iters - 1, 0, False, True)
        for b in range(2):
            _, _, srows, drows, _, _, osem = bufs[b]
            pltpu.make_async_copy(srows, osrc_hbm.at[pl.ds(0, _C)],
                                  osem).wait()
            pltpu.make_async_copy(drows, odst_hbm.at[pl.ds(0, _C)],
                                  osem).wait()

    return k(src_na, dst_na, sidx, didx)


def _sc_scatter(didx, evx, n):
    e = evx.shape[1]
    ng = evx.shape[0] // 8
    cc = 1280
    iters = e // cc
    groups = cc // 16
    zc = 16000
    zi = (n * 8) // zc
    mesh = plsc.VectorSubcoreMesh(core_axis_name="c", subcore_axis_name="s")

    assert iters % 2 == 0 and groups % 2 == 0

    @functools.partial(
        pl.kernel,
        out_type=jax.ShapeDtypeStruct((ng, 8, n), jnp.float32),
        mesh=mesh,
        compiler_params=pltpu.CompilerParams(needs_layout_passes=False),
        scratch_types=[
            pltpu.VMEM((cc,), jnp.int32),
            pltpu.VMEM((cc,), jnp.int32),
            pltpu.VMEM((8, cc), jnp.float32),
            pltpu.VMEM((8, cc), jnp.float32),
        ] + [pltpu.VMEM((n,), jnp.float32) for _ in range(8)] + [
            pltpu.SemaphoreType.DMA,
            pltpu.SemaphoreType.DMA,
        ],
    )
    def k(didx_hbm, evx_hbm, out_hbm, idxv0, idxv1, vbuf0, vbuf1,
          a0, a1, a2, a3, a4, a5, a6, a7, sem0, sem1):
        accs = (a0, a1, a2, a3, a4, a5, a6, a7)
        c = lax.axis_index("c")
        s = lax.axis_index("s")
        w = s * _NC + c

        @pl.when(w < ng)
        def _():
            zeros16 = jnp.zeros((16,), jnp.float32)

            def initb(j, carry):
                for kk in range(8):
                    accs[kk][pl.ds(j * 16, 16)] = zeros16
                return carry

            lax.fori_loop(0, n // 16, initb, 0)

            def start(i, idxv, vbuf, sem):
                base = i * cc
                pltpu.async_copy(didx_hbm.at[pl.ds(base, cc)], idxv, sem)
                pltpu.async_copy(evx_hbm.at[pl.ds(w * 8, 8),
                                            pl.ds(base, cc)], vbuf, sem)

            def wait(idxv, vbuf, sem):
                pltpu.make_async_copy(didx_hbm.at[pl.ds(0, cc)], idxv,
                                      sem).wait()
                pltpu.make_async_copy(evx_hbm.at[pl.ds(0, 8), pl.ds(0, cc)],
                                      vbuf, sem).wait()

            def process(idxv, vbuf):
                def grp(g2, carry2):
                    for u in range(2):
                        g16 = (g2 * 2 + u) * 16
                        nidx = idxv[pl.ds(g16, 16)]
                        for kk in range(8):
                            plsc.addupdate_scatter(
                                accs[kk], [nidx], vbuf[kk, pl.ds(g16, 16)])
                    return carry2

                lax.fori_loop(0, groups // 2, grp, 0)

            start(0, idxv0, vbuf0, sem0)

            def body(i2, carry):
                i = i2 * 2
                wait(idxv0, vbuf0, sem0)

                @pl.when(i + 1 < iters)
                def _():
                    start(i + 1, idxv1, vbuf1, sem1)

                process(idxv0, vbuf0)

                @pl.when(i + 1 < iters)
                def _():
                    wait(idxv1, vbuf1, sem1)

                    @pl.when(i + 2 < iters)
                    def _():
                        start(i + 2, idxv0, vbuf0, sem0)

                    process(idxv1, vbuf1)

                return carry

            lax.fori_loop(0, iters // 2, body, 0)

            for kk in range(8):
                pltpu.sync_copy(accs[kk], out_hbm.at[w, kk])

    return k(didx, evx)


def _gelu(x):
    return 0.5 * x * (1.0 + lax.erf(x * (1.0 / math.sqrt(2.0))))


def _full_spec(shape):
    return pl.BlockSpec(shape, lambda i: tuple(0 for _ in shape))


def _tc_fused(srows, drows, ea, wb1s, wb1e, wb1d, b1, wb2, b2, wb3, b3,
              wv1s, wv1e, c1, wv2, c2, wv3, c3, heads):
    e, h = ea.shape
    eb = 1280
    grid = e // eb
    d = h // heads
    inv = 1.0 / math.sqrt(d)

    def body(xs_ref, xd_ref, xe_ref, wb1s_r, wb1e_r, wb1d_r, b1_r, wb2_r,
             b2_r, wb3_r, b3_r, wv1s_r, wv1e_r, c1_r, wv2_r, c2_r, wv3_r,
             c3_r, evx_ref):
        xs = xs_ref[...]
        xd = xd_ref[...]
        xe = xe_ref[...]
        hb = xs @ wb1s_r[...] + xe @ wb1e_r[...] + xd @ wb1d_r[...] + b1_r[...]
        hb = jnp.maximum(hb, 0.0)
        hb = jnp.maximum(hb @ wb2_r[...] + b2_r[...], 0.0)
        w = (hb @ wb3_r[...] + b3_r[...]) * inv
        ex = jnp.exp(jnp.clip(w, -60.0, 60.0))
        hv = _gelu(xs @ wv1s_r[...] + xe @ wv1e_r[...] + c1_r[...])
        hv = _gelu(hv @ wv2_r[...] + c2_r[...])
        v = hv @ wv3_r[...] + c3_r[...]
        hr = lax.broadcasted_iota(jnp.int32, (heads, h), 0)
        lh = lax.broadcasted_iota(jnp.int32, (heads, h), 1) // d
        ev = v * (ex @ jnp.where(hr == lh, 1.0, 0.0))
        i0 = lax.broadcasted_iota(jnp.int32, (heads, 8), 0)
        i1 = lax.broadcasted_iota(jnp.int32, (heads, 8), 1)
        ep8 = ex @ jnp.where(i0 == i1, 1.0, 0.0)
        evx_ref[...] = jnp.concatenate([ev.T, ep8.T], axis=0)

    weights = [wb1s, wb1e, wb1d, b1, wb2, b2, wb3, b3,
               wv1s, wv1e, c1, wv2, c2, wv3, c3]
    return pl.pallas_call(
        body,
        grid=(grid,),
        in_specs=[pl.BlockSpec((eb, h), lambda i: (i, 0))] * 3
                 + [_full_spec(x.shape) for x in weights],
        out_specs=pl.BlockSpec((h + 8, eb), lambda i: (0, i)),
        out_shape=jax.ShapeDtypeStruct((h + 8, e), jnp.float32),
    )(srows, drows, ea, *weights)


def _tc_final(slabs, wo, heads):
    n = slabs.shape[1]
    h = wo.shape[0]
    d = h // heads

    def body(sl_ref, wo_ref, o_ref):
        allr = sl_ref[...]
        numt = allr[:h, :]
        s8t = allr[h:h + 8, :]
        lr = lax.broadcasted_iota(jnp.int32, (h, 8), 0) // d
        jr = lax.broadcasted_iota(jnp.int32, (h, 8), 1)
        srept = jnp.where(jr == lr, 1.0, 0.0) @ s8t + 1e-16
        o_ref[...] = (wo_ref[...] @ (numt / srept)).T

    return pl.pallas_call(
        body,
        grid=(1,),
        in_specs=[_full_spec((h + 8, n)), _full_spec((h, h))],
        out_specs=pl.BlockSpec((n, h), lambda i: (0, 0)),
        out_shape=jax.ShapeDtypeStruct((n, h), jnp.float32),
    )(slabs, wo)


def kernel(src_na, dst_na, ea, edge_idx, Wv1, bv1, Wv2, bv2, Wv3, bv3,
           Wb1, bb1, Wb2, bb2, Wb3, bb3, Wo):
    n, h = src_na.shape
    heads = Wb3.shape[0]
    dst_idx = edge_idx[0]
    src_idx = edge_idx[1]

    srows, drows = _sc_gather(src_na, dst_na, src_idx, dst_idx)

    evx = _tc_fused(
        srows, drows, ea,
        Wb1[:, :h].T, Wb1[:, h:2 * h].T, Wb1[:, 2 * h:].T, bb1.reshape(1, -1),
        Wb2.T, bb2.reshape(1, -1), Wb3.T, bb3.reshape(1, -1),
        Wv1[:, :h].T, Wv1[:, h:].T, bv1.reshape(1, -1),
        Wv2.T, bv2.reshape(1, -1), Wv3.T, bv3.reshape(1, -1), heads)

    slabs = _sc_scatter(dst_idx, evx, n).reshape(-1, n)
    return _tc_final(slabs, Wo, heads)
